# baseline (speedup 1.0000x reference)
import functools

import jax
import jax.numpy as jnp
from jax import lax
from jax.experimental import pallas as pl
from jax.experimental.pallas import tpu as pltpu
from jax.experimental.pallas import tpu_sc as plsc

_STATIC = 26
_CONT = 13
_ED = 64
_BATCH = 16384
_NPAIR = _STATIC // 2
_OUTW = (_STATIC + _CONT) * _ED

_NC, _NS = 2, 16
_NW = _NC * _NS
_RPW = _BATCH // _NW
_C = 16
_NCHUNK = _RPW // _C


def _sc_body(si_ev_ref, si_od_ref, tab2_ref, out_ref, ev_v, od_v, pidx_v, gbuf, sem):
    w = lax.axis_index("s") * _NC + lax.axis_index("c")
    base = w * _RPW
    pltpu.sync_copy(si_ev_ref.at[:, pl.ds(base, _RPW)], ev_v)
    pltpu.sync_copy(si_od_ref.at[:, pl.ds(base, _RPW)], od_v)

    def compute_pidx(g, carry):
        r0 = g * 16
        for k in range(_NPAIR):
            ev = ev_v[k, pl.ds(r0, 16)]
            od = od_v[k, pl.ds(r0, 16)]
            pidx_v[pl.ds(k * _RPW + r0, 16)] = ev * _STATIC + od
        return carry

    lax.fori_loop(0, _RPW // 16, compute_pidx, 0)

    def chunk(c, carry):
        r0 = c * _C
        copies = []
        for k in range(_NPAIR):
            copies.append(pltpu.async_copy(
                tab2_ref.at[pidx_v.at[pl.ds(k * _RPW + r0, _C)]],
                gbuf.at[k], sem))
        for cp in copies:
            cp.wait()
        copies = []
        for k in range(_NPAIR):
            copies.append(pltpu.async_copy(
                gbuf.at[k],
                out_ref.at[pl.ds(base + r0, _C), pl.ds(128 * k, 128)],
                sem))
        for cp in copies:
            cp.wait()
        return carry

    lax.fori_loop(0, _NCHUNK, chunk, 0)


_sc_gather = functools.partial(
    pl.kernel,
    out_type=jax.ShapeDtypeStruct((_BATCH, _OUTW), jnp.float32),
    mesh=plsc.VectorSubcoreMesh(
        core_axis_name="c", subcore_axis_name="s", num_cores=_NC, num_subcores=_NS),
    scratch_types=[
        pltpu.VMEM((_NPAIR, _RPW), jnp.int32),
        pltpu.VMEM((_NPAIR, _RPW), jnp.int32),
        pltpu.VMEM((_NPAIR * _RPW,), jnp.int32),
        pltpu.VMEM((_NPAIR, _C, 128), jnp.float32),
        pltpu.SemaphoreType.DMA,
    ],
)(_sc_body)


_R = 256


def _tc_fill_body(buf_ref, ci_ref, wt_ref, b_ref, out_ref, acc_ref, sem):
    i = pl.program_id(0)
    acc_ref[...] = (
        jnp.dot(ci_ref[...], wt_ref[...], preferred_element_type=jnp.float32)
        + b_ref[...])
    pltpu.async_copy(
        acc_ref,
        out_ref.at[pl.ds(i * _R, _R), pl.ds(_STATIC * _ED, _CONT * _ED)],
        sem).wait()


def kernel(static_input, continuous_input, table, W, b):
    pr = jnp.arange(_STATIC * _STATIC, dtype=jnp.int32)
    tab2 = jnp.concatenate(
        [table[pr // _STATIC], table[pr % _STATIC]], axis=1)
    si_ev = static_input[:, 0::2].T
    si_od = static_input[:, 1::2].T
    wt = W.T
    b2 = b.reshape(1, _CONT * _ED)

    out = _sc_gather(si_ev, si_od, tab2)

    out = pl.pallas_call(
        _tc_fill_body,
        grid=(_BATCH // _R,),
        in_specs=[
            pl.BlockSpec(memory_space=pl.ANY),
            pl.BlockSpec((_R, _CONT), lambda i: (i, 0)),
            pl.BlockSpec((_CONT, _CONT * _ED), lambda i: (0, 0)),
            pl.BlockSpec((1, _CONT * _ED), lambda i: (0, 0)),
        ],
        out_specs=pl.BlockSpec(memory_space=pl.ANY),
        out_shape=jax.ShapeDtypeStruct((_BATCH, _OUTW), jnp.float32),
        input_output_aliases={0: 0},
        scratch_shapes=[
            pltpu.VMEM((_R, _CONT * _ED), jnp.float32),
            pltpu.SemaphoreType.DMA,
        ],
    )(out, continuous_input, wt, b2)
    return out
